# baseline (speedup 1.0000x reference)
import functools

import jax
import jax.numpy as jnp
import numpy as np
from jax import lax
from jax.experimental import pallas as pl
from jax.experimental.pallas import tpu as pltpu
from jax.experimental.pallas import tpu_sc as plsc

B = 4096
N = 128
NP = N + 1
D = N * (N + 1) // 2
NN = N * N
NCHUNK = D // 16
UNROLL = 8
DEPTH = 4

_NUM_CORES = 2
_NUM_SUBCORES = 16
_NUM_WORKERS = _NUM_CORES * _NUM_SUBCORES
ROWS_PER_WORKER = B // _NUM_WORKERS


def _scatter_table() -> np.ndarray:
    iu, ju = np.triu_indices(N)
    return np.concatenate([iu.astype(np.int32), ju.astype(np.int32)])


_mesh = plsc.VectorSubcoreMesh(core_axis_name="c", subcore_axis_name="s")


@functools.partial(
    pl.kernel,
    out_type=jax.ShapeDtypeStruct((B * N, N), jnp.float32),
    mesh=_mesh,
    compiler_params=pltpu.CompilerParams(needs_layout_passes=False,
                                         use_tc_tiling_on_sc=False),
    scratch_types=(
        [pltpu.VMEM((2 * D,), jnp.int32)]
        + [pltpu.VMEM((1, D), jnp.float32) for _ in range(DEPTH)]
        + [pltpu.VMEM((N, NP), jnp.float32) for _ in range(DEPTH)]
        + [pltpu.SemaphoreType.DMA for _ in range(2 * DEPTH)]
    ),
)
def _unvec_kernel(x_hbm, idx_hbm, out_hbm, idx_v, *bufs):
    in_bufs = bufs[:DEPTH]
    out_bufs = bufs[DEPTH:2 * DEPTH]
    in_sems = bufs[2 * DEPTH:3 * DEPTH]
    out_sems = bufs[3 * DEPTH:4 * DEPTH]

    wid = lax.axis_index("s") * _NUM_CORES + lax.axis_index("c")
    base = wid * ROWS_PER_WORKER
    pltpu.sync_copy(idx_hbm, idx_v)

    def start_in(g, s):
        b = base + g
        pltpu.async_copy(x_hbm.at[pl.ds(b, 1), :], in_bufs[s], in_sems[s])

    def wait_in(s):
        pltpu.make_async_copy(
            x_hbm.at[pl.ds(base, 1), :], in_bufs[s], in_sems[s]).wait()

    def start_out(g, s):
        b = base + g
        pltpu.async_copy(out_bufs[s].at[:, pl.ds(0, N)],
                         out_hbm.at[pl.ds(b * N, N)], out_sems[s])

    def wait_out(s):
        pltpu.make_async_copy(
            out_bufs[s].at[:, pl.ds(0, N)],
            out_hbm.at[pl.ds(base * N, N)], out_sems[s]).wait()

    for p in range(DEPTH - 1):
        start_in(p, p)

    def outer(g4, carry):
        for s in range(DEPTH):
            g = g4 * DEPTH + s
            wait_in(s)

            @pl.when(g + DEPTH - 1 < ROWS_PER_WORKER)
            def _():
                start_in(g + DEPTH - 1, (s + DEPTH - 1) % DEPTH)

            @pl.when(g >= DEPTH)
            def _():
                wait_out(s)

            src = in_bufs[s]
            dst = out_bufs[s]

            @plsc.parallel_loop(0, NCHUNK, 1, unroll=UNROLL)
            def chunk(k, src=src, dst=dst):
                off = k * 16
                a = idx_v[pl.ds(off, 16)]
                b2 = idx_v[pl.ds(D + off, 16)]
                v = src[0, pl.ds(off, 16)]
                plsc.store_scatter(dst, [a, b2], v)
                plsc.store_scatter(dst, [b2, a], v)

            start_out(g, s)
        return carry

    lax.fori_loop(0, ROWS_PER_WORKER // DEPTH, outer, 0, unroll=False)
    for s in range(DEPTH):
        wait_out(s)


def kernel(input):
    idx = jnp.asarray(_scatter_table())
    out = _unvec_kernel(input, idx)
    return out.reshape(B, N, N)
